# baseline (speedup 1.0000x reference)
import functools

import jax
import jax.numpy as jnp
from jax import lax
from jax.experimental import pallas as pl
from jax.experimental.pallas import tpu as pltpu
from jax.experimental.pallas import tpu_sc as plsc

N = 10000
E = 320000
D = 128
H = 256
OUT = 128

EBLK = 2560

NC = 2
NS = 16
NW = NC * NS
L = 16

_SC_MESH = dict(core_axis_name="c", subcore_axis_name="s")


def _wid():
    return lax.axis_index("s") * NC + lax.axis_index("c")


EPW = E // NW
GCBS = 200


def _edge_gather_body(u_hbm, pw_hbm, src_hbm, dst_hbm, r_hbm,
                      sidx, didx, ubuf, pbuf, sem1, sem2):
    base = _wid() * EPW

    def chunk(i, carry):
        off = base + i * GCBS
        pltpu.sync_copy(src_hbm.at[pl.ds(off, GCBS)], sidx)
        pltpu.sync_copy(dst_hbm.at[pl.ds(off, GCBS)], didx)
        cu = pltpu.async_copy(u_hbm.at[sidx], ubuf, sem1)
        cp = pltpu.async_copy(pw_hbm.at[didx], pbuf, sem2)
        cu.wait()
        cp.wait()

        def row(r, c2):
            for c in range(H // L):
                s = pl.ds(c * L, L)
                ubuf[r, s] = jnp.maximum(ubuf[r, s] - pbuf[r, s], 0.0)
            return c2

        lax.fori_loop(0, GCBS, row, 0)
        pltpu.sync_copy(ubuf, r_hbm.at[pl.ds(off, GCBS)])
        return carry

    lax.fori_loop(0, EPW // GCBS, chunk, 0)


def _edge_gather(u, pw, src, dst):
    return pl.kernel(
        _edge_gather_body,
        out_type=jax.ShapeDtypeStruct((E, H), jnp.float32),
        mesh=plsc.VectorSubcoreMesh(**_SC_MESH),
        scratch_types=[
            pltpu.VMEM((GCBS,), jnp.int32),
            pltpu.VMEM((GCBS,), jnp.int32),
            pltpu.VMEM((GCBS, H), jnp.float32),
            pltpu.VMEM((GCBS, H), jnp.float32),
            pltpu.SemaphoreType.DMA,
            pltpu.SemaphoreType.DMA,
        ],
    )(u, pw, src, dst)


SCBS = 200


def _segsum_run(z_hbm, out_hbm, src_hbm, dst_hbm, acc_sh, sidx, didx, buf, sem):
    sid = lax.axis_index("s")
    ept = E // NS

    @pl.when(sid < 10)
    def _():
        pltpu.sync_copy(z_hbm.at[pl.ds(sid * 1000, 1000)],
                        acc_sh.at[pl.ds(sid * 1000, 1000)])

    plsc.subcore_barrier()

    def chunk(i, carry):
        off = sid * ept + i * SCBS
        pltpu.sync_copy(src_hbm.at[pl.ds(off, SCBS)], sidx)
        pltpu.sync_copy(dst_hbm.at[pl.ds(off, SCBS)], didx)
        pltpu.async_copy(z_hbm.at[sidx], buf, sem).wait()
        pltpu.async_copy(buf, acc_sh.at[didx], sem, add=True).wait()
        return carry

    lax.fori_loop(0, ept // SCBS, chunk, 0)
    plsc.subcore_barrier()

    @pl.when(sid < 10)
    def _():
        pltpu.sync_copy(acc_sh.at[pl.ds(sid * 1000, 1000)],
                        out_hbm.at[pl.ds(sid * 1000, 1000)])


def _segsum_body(zl_hbm, zr_hbm, src_hbm, dst_hbm, outl_hbm, outr_hbm,
                 acc_sh, sidx, didx, buf, sem):
    c = lax.axis_index("c")

    @pl.when(c == 0)
    def _():
        _segsum_run(zl_hbm, outl_hbm, src_hbm, dst_hbm, acc_sh, sidx, didx, buf, sem)

    @pl.when(c == 1)
    def _():
        _segsum_run(zr_hbm, outr_hbm, src_hbm, dst_hbm, acc_sh, sidx, didx, buf, sem)


def _segsum(zl, zr, src, dst, hc2):
    return pl.kernel(
        _segsum_body,
        out_type=(jax.ShapeDtypeStruct((N, hc2), jnp.float32),
                  jax.ShapeDtypeStruct((N, hc2), jnp.float32)),
        mesh=plsc.VectorSubcoreMesh(**_SC_MESH),
        scratch_types=[
            pltpu.VMEM_SHARED((N, hc2), jnp.float32),
            pltpu.VMEM((SCBS,), jnp.int32),
            pltpu.VMEM((SCBS,), jnp.int32),
            pltpu.VMEM((SCBS, hc2), jnp.float32),
            pltpu.SemaphoreType.DMA,
        ],
    )(zl, zr, src, dst)


def _segsum_es_body(z_hbm, src_hbm, dst_hbm, out_hbm, acc_sh, sidx, didx, buf, sem):
    c = lax.axis_index("c")
    sid = lax.axis_index("s")

    @pl.when(sid < 10)
    def _():
        pltpu.sync_copy(z_hbm.at[pl.ds(sid * 1000, 1000)],
                        acc_sh.at[pl.ds(sid * 1000, 1000)])

    plsc.subcore_barrier()

    def chunk(i, carry):
        off = (c * NS + sid) * EPW + i * SCBS
        pltpu.sync_copy(src_hbm.at[pl.ds(off, SCBS)], sidx)
        pltpu.sync_copy(dst_hbm.at[pl.ds(off, SCBS)], didx)
        pltpu.async_copy(z_hbm.at[sidx], buf, sem).wait()
        pltpu.async_copy(buf, acc_sh.at[didx], sem, add=True).wait()
        return carry

    lax.fori_loop(0, EPW // SCBS, chunk, 0)
    plsc.subcore_barrier()

    @pl.when(sid < 10)
    def _():
        pltpu.sync_copy(acc_sh.at[pl.ds(sid * 1000, 1000)],
                        out_hbm.at[c, pl.ds(sid * 1000, 1000)])


def _segsum_es(z, src, dst, w):
    parts = pl.kernel(
        _segsum_es_body,
        out_type=jax.ShapeDtypeStruct((NC, N, w), jnp.float32),
        mesh=plsc.VectorSubcoreMesh(**_SC_MESH),
        scratch_types=[
            pltpu.VMEM_SHARED((N, w), jnp.float32),
            pltpu.VMEM((SCBS,), jnp.int32),
            pltpu.VMEM((SCBS,), jnp.int32),
            pltpu.VMEM((SCBS, w), jnp.float32),
            pltpu.SemaphoreType.DMA,
        ],
    )(z, src, dst)
    return parts[0] + parts[1] - z


NPT = 320
NPAD = NW * NPT
EPWP = EPW + 32 * 16 + 16
_SENT = jnp.int32((NPT << 19))
_M19 = jnp.int32((1 << 19) - 1)


def _route_body(dst_hbm, ebuf_hbm, meta_hbm, dchunk, ebuf, stage, metavm,
                cntv, startsv, ptrv):
    w = _wid()
    lidx = lax.iota(jnp.int32, L)
    z16 = jnp.zeros((L,), jnp.int32)
    pltpu.sync_copy(dst_hbm.at[pl.ds(w * EPW, EPW)], dchunk)

    def fill(i, carry):
        ebuf[pl.ds(i * L, L)] = z16
        return carry

    lax.fori_loop(0, EPWP // L, fill, 0)

    def zero(o, carry):
        cntv[o] = 0
        stage[pl.ds(o * L, L)] = z16
        return carry

    lax.fori_loop(0, 32, zero, 0)

    def count(g, carry):
        dv = dchunk[pl.ds(g * L, L)]
        for j in range(L):
            o = (dv[j] * 6554) >> 21
            cntv[o] = cntv[o] + 1
        return carry

    lax.fori_loop(0, EPW // L, count, 0)

    def prefix(o, pos):
        c = cntv[o]
        startsv[o] = pos
        ptrv[o] = 0
        return (pos + c + 15) & (-16)

    lax.fori_loop(0, 32, prefix, 0)

    def place(g, carry):
        dv = dchunk[pl.ds(g * L, L)]
        for j in range(L):
            d = dv[j]
            o = (d * 6554) >> 21
            dloc = d - o * NPT
            word = (dloc << 19) | (w * EPW + g * L + j)
            cnt = ptrv[o]
            sv = stage[pl.ds(o * L, L)]
            nv = jnp.where(lidx == (cnt & 15), word, sv)
            stage[pl.ds(o * L, L)] = nv

            @pl.when((cnt & 15) == 15)
            def _():
                ebuf[pl.ds(startsv[o] + cnt - 15, L)] = nv

            ptrv[o] = cnt + 1
        return carry

    lax.fori_loop(0, EPW // L, place, 0)

    def flush(o, carry):
        cnt = ptrv[o]

        @pl.when((cnt & 15) > 0)
        def _():
            ebuf[pl.ds(startsv[o] + (cnt & (-16)), L)] = stage[pl.ds(o * L, L)]

        return carry

    lax.fori_loop(0, 32, flush, 0)

    s0 = z16
    s1 = z16
    c0 = z16
    c1 = z16
    for o in range(16):
        s0 = jnp.where(lidx == o, startsv[o], s0)
        s1 = jnp.where(lidx == o, startsv[o + 16], s1)
        c0 = jnp.where(lidx == o, cntv[o], c0)
        c1 = jnp.where(lidx == o, cntv[o + 16], c1)
    metavm[pl.ds(0, L)] = s0
    metavm[pl.ds(L, L)] = s1
    metavm[pl.ds(2 * L, L)] = c0
    metavm[pl.ds(3 * L, L)] = c1

    pltpu.sync_copy(ebuf, ebuf_hbm.at[pl.ds(w * EPWP, EPWP)])
    pltpu.sync_copy(metavm, meta_hbm.at[pl.ds(w * 64, 64)])


def _segmax_body(h_hbm, selfh_hbm, ebuf_hbm, m5_hbm, out_hbm,
                 agg, tb, wbuf, idxbuf, rowbuf, sem):
    w = _wid()
    lo = w * NPT
    pltpu.sync_copy(selfh_hbm.at[pl.ds(lo, NPT)], agg)

    def srctile(t, carry):
        pltpu.sync_copy(m5_hbm.at[pl.ds((t * NW + w) * L, L)], tb)
        tv = tb[pl.ds(0, L)]
        st = pl.multiple_of(tv[0], L)
        ln = tv[1]

        def batch(b, c2):
            off = st + b * L
            pltpu.sync_copy(ebuf_hbm.at[pl.ds(t * EPWP + off, L)], wbuf)
            wv = wbuf[pl.ds(0, L)]
            idxbuf[pl.ds(0, L)] = wv & _M19
            pltpu.async_copy(h_hbm.at[idxbuf], rowbuf, sem).wait()
            for j in range(L):
                dloc = wv[j] >> 19

                @pl.when(b * L + j < ln)
                def _():
                    for c in range(H // L):
                        s = pl.ds(c * L, L)
                        agg[dloc, s] = jnp.maximum(agg[dloc, s], rowbuf[j, s])
            return c2

        lax.fori_loop(0, (ln + L - 1) // L, batch, 0)
        return carry

    lax.fori_loop(0, NW, srctile, 0)
    pltpu.sync_copy(agg, out_hbm.at[pl.ds(lo, NPT)])


def _segmax(h, selfh_pad, dst):
    ebufs, meta = pl.kernel(
        _route_body,
        out_type=(jax.ShapeDtypeStruct((NW * EPWP,), jnp.int32),
                  jax.ShapeDtypeStruct((NW * 64,), jnp.int32)),
        mesh=plsc.VectorSubcoreMesh(**_SC_MESH),
        scratch_types=[
            pltpu.VMEM((EPW,), jnp.int32),
            pltpu.VMEM((EPWP,), jnp.int32),
            pltpu.VMEM((32 * L,), jnp.int32),
            pltpu.VMEM((64,), jnp.int32),
            pltpu.SMEM((32,), jnp.int32),
            pltpu.SMEM((32,), jnp.int32),
            pltpu.SMEM((32,), jnp.int32),
        ],
    )(dst)
    m = meta.reshape(NW, 2, 32)
    m5 = jnp.zeros((NW * NW, L), jnp.int32)
    m5 = m5.at[:, 0].set(m[:, 0, :].reshape(-1))
    m5 = m5.at[:, 1].set(m[:, 1, :].reshape(-1))
    return pl.kernel(
        _segmax_body,
        out_type=jax.ShapeDtypeStruct((NPAD, H), jnp.float32),
        mesh=plsc.VectorSubcoreMesh(**_SC_MESH),
        scratch_types=[
            pltpu.VMEM((NPT, H), jnp.float32),
            pltpu.VMEM((L,), jnp.int32),
            pltpu.VMEM((L,), jnp.int32),
            pltpu.VMEM((L,), jnp.int32),
            pltpu.VMEM((L, H), jnp.float32),
            pltpu.SemaphoreType.DMA,
        ],
    )(h, selfh_pad, ebufs, m5.reshape(-1))


DCBS = 2000


def _deg_body(dst_hbm, out_hbm, cnt, dchunk):
    w = _wid()

    def z16(i, carry):
        cnt[pl.ds(i * L, L)] = jnp.zeros((L,), jnp.float32)
        return carry

    lax.fori_loop(0, N // L + 1, z16, 0)
    ones = jnp.ones((L,), jnp.float32)

    def chunk(i, carry):
        off = w * EPW + i * DCBS
        pltpu.sync_copy(dst_hbm.at[pl.ds(off, DCBS)], dchunk)

        def grp(j, cc):
            idx = dchunk[pl.ds(j * L, L)]
            plsc.addupdate_scatter(cnt, [idx], ones)
            return cc

        lax.fori_loop(0, DCBS // L, grp, 0)
        return carry

    lax.fori_loop(0, EPW // DCBS, chunk, 0)
    pltpu.sync_copy(cnt, out_hbm.at[w])


def _deg_counts(dst):
    return pl.kernel(
        _deg_body,
        out_type=jax.ShapeDtypeStruct((NW, N + L), jnp.float32),
        mesh=plsc.VectorSubcoreMesh(**_SC_MESH),
        scratch_types=[
            pltpu.VMEM((N + L,), jnp.float32),
            pltpu.VMEM((DCBS,), jnp.int32),
        ],
    )(dst)


def _edge_mlp_body(a_ref, w_ref, b_ref, o_ref):
    a = jnp.maximum(a_ref[...], 0.0)
    o_ref[...] = jnp.dot(a, w_ref[...], preferred_element_type=jnp.float32) + b_ref[...]


def _edge_mlp(a, w, b):
    e = a.shape[0]
    grid = e // EBLK
    return pl.pallas_call(
        _edge_mlp_body,
        grid=(grid,),
        in_specs=[
            pl.BlockSpec((EBLK, H), lambda i: (i, 0)),
            pl.BlockSpec((H, H), lambda i: (0, 0)),
            pl.BlockSpec((1, H), lambda i: (0, 0)),
        ],
        out_specs=pl.BlockSpec((EBLK, H), lambda i: (i, 0)),
        out_shape=jax.ShapeDtypeStruct((e, H), jnp.float32),
    )(a, w, b.reshape(1, H))


def kernel(x, pos, edge_index, lW1, lb1, lW2, lb2, gW1, gb1, gW2, gb2, gW3, gb3, cW1, cb1, cW2, cb2):
    n = x.shape[0]
    src = edge_index[0]
    dst = edge_index[1]

    xw = x @ lW1[:D] + lb1
    pw = pos @ lW1[D:]
    u = xw + pw

    r_e = _edge_gather(u, pw, src, dst)
    h_e = _edge_mlp(r_e, lW2, lb2)
    self_h = jnp.maximum(xw, 0.0) @ lW2 + lb2

    self_h_pad = jnp.pad(self_h, ((0, NPAD - N), (0, 0)))
    agg = _segmax(h_e, self_h_pad, dst)[:N]

    g = jnp.maximum(agg @ gW1 + gb1, 0.0)
    g = jnp.maximum(g @ gW2 + gb2, 0.0)
    h0 = g @ gW3 + gb3

    deg = jax.ops.segment_sum(jnp.ones((E,), jnp.float32), dst, num_segments=n) + 1.0
    dis = jax.lax.rsqrt(deg)[:, None]

    z1 = dis * (h0 @ cW1)
    acc1l, acc1r = _segsum(z1[:, : H // 2], z1[:, H // 2 :], src, dst, H // 2)
    h1 = jnp.maximum(dis * jnp.concatenate([acc1l, acc1r], axis=1) + cb1, 0.0)

    z2 = dis * (h1 @ cW2)
    acc2 = _segsum_es(z2, src, dst, OUT)
    h2 = jnp.maximum(dis * acc2 + cb2, 0.0)
    return h2


# speedup vs baseline: 1.0127x; 1.0127x over previous
import functools

import jax
import jax.numpy as jnp
from jax import lax
from jax.experimental import pallas as pl
from jax.experimental.pallas import tpu as pltpu
from jax.experimental.pallas import tpu_sc as plsc

N = 10000
E = 320000
D = 128
H = 256
OUT = 128

EBLK = 2560

NC = 2
NS = 16
NW = NC * NS
L = 16

_SC_MESH = dict(core_axis_name="c", subcore_axis_name="s")


def _wid():
    return lax.axis_index("s") * NC + lax.axis_index("c")


EPW = E // NW
GCBS = 200


def _edge_gather_body(u_hbm, pw_hbm, src_hbm, dst_hbm, r_hbm,
                      sidx, didx, ubuf, pbuf, sem1, sem2):
    base = _wid() * EPW

    def chunk(i, carry):
        off = base + i * GCBS
        pltpu.sync_copy(src_hbm.at[pl.ds(off, GCBS)], sidx)
        pltpu.sync_copy(dst_hbm.at[pl.ds(off, GCBS)], didx)
        cu = pltpu.async_copy(u_hbm.at[sidx], ubuf, sem1)
        cp = pltpu.async_copy(pw_hbm.at[didx], pbuf, sem2)
        cu.wait()
        cp.wait()

        def row(r, c2):
            for c in range(H // L):
                s = pl.ds(c * L, L)
                ubuf[r, s] = jnp.maximum(ubuf[r, s] - pbuf[r, s], 0.0)
            return c2

        lax.fori_loop(0, GCBS, row, 0)
        pltpu.sync_copy(ubuf, r_hbm.at[pl.ds(off, GCBS)])
        return carry

    lax.fori_loop(0, EPW // GCBS, chunk, 0)


def _edge_gather(u, pw, src, dst):
    return pl.kernel(
        _edge_gather_body,
        out_type=jax.ShapeDtypeStruct((E, H), jnp.float32),
        mesh=plsc.VectorSubcoreMesh(**_SC_MESH),
        scratch_types=[
            pltpu.VMEM((GCBS,), jnp.int32),
            pltpu.VMEM((GCBS,), jnp.int32),
            pltpu.VMEM((GCBS, H), jnp.float32),
            pltpu.VMEM((GCBS, H), jnp.float32),
            pltpu.SemaphoreType.DMA,
            pltpu.SemaphoreType.DMA,
        ],
    )(u, pw, src, dst)


SCBS = 200


def _segsum_run(z_hbm, out_hbm, src_hbm, dst_hbm, acc_sh, sidx, didx, buf, sem):
    sid = lax.axis_index("s")
    ept = E // NS

    @pl.when(sid < 10)
    def _():
        pltpu.sync_copy(z_hbm.at[pl.ds(sid * 1000, 1000)],
                        acc_sh.at[pl.ds(sid * 1000, 1000)])

    plsc.subcore_barrier()

    def chunk(i, carry):
        off = sid * ept + i * SCBS
        pltpu.sync_copy(src_hbm.at[pl.ds(off, SCBS)], sidx)
        pltpu.sync_copy(dst_hbm.at[pl.ds(off, SCBS)], didx)
        pltpu.async_copy(z_hbm.at[sidx], buf, sem).wait()
        pltpu.async_copy(buf, acc_sh.at[didx], sem, add=True).wait()
        return carry

    lax.fori_loop(0, ept // SCBS, chunk, 0)
    plsc.subcore_barrier()

    @pl.when(sid < 10)
    def _():
        pltpu.sync_copy(acc_sh.at[pl.ds(sid * 1000, 1000)],
                        out_hbm.at[pl.ds(sid * 1000, 1000)])


def _segsum_body(zl_hbm, zr_hbm, src_hbm, dst_hbm, outl_hbm, outr_hbm,
                 acc_sh, sidx, didx, buf, sem):
    c = lax.axis_index("c")

    @pl.when(c == 0)
    def _():
        _segsum_run(zl_hbm, outl_hbm, src_hbm, dst_hbm, acc_sh, sidx, didx, buf, sem)

    @pl.when(c == 1)
    def _():
        _segsum_run(zr_hbm, outr_hbm, src_hbm, dst_hbm, acc_sh, sidx, didx, buf, sem)


def _segsum(zl, zr, src, dst, hc2):
    return pl.kernel(
        _segsum_body,
        out_type=(jax.ShapeDtypeStruct((N, hc2), jnp.float32),
                  jax.ShapeDtypeStruct((N, hc2), jnp.float32)),
        mesh=plsc.VectorSubcoreMesh(**_SC_MESH),
        scratch_types=[
            pltpu.VMEM_SHARED((N, hc2), jnp.float32),
            pltpu.VMEM((SCBS,), jnp.int32),
            pltpu.VMEM((SCBS,), jnp.int32),
            pltpu.VMEM((SCBS, hc2), jnp.float32),
            pltpu.SemaphoreType.DMA,
        ],
    )(zl, zr, src, dst)


def _segsum_es_body(z_hbm, src_hbm, dst_hbm, out_hbm, acc_sh, sidx, didx, buf, sem):
    c = lax.axis_index("c")
    sid = lax.axis_index("s")

    @pl.when(sid < 10)
    def _():
        pltpu.sync_copy(z_hbm.at[pl.ds(sid * 1000, 1000)],
                        acc_sh.at[pl.ds(sid * 1000, 1000)])

    plsc.subcore_barrier()

    def chunk(i, carry):
        off = (c * NS + sid) * EPW + i * SCBS
        pltpu.sync_copy(src_hbm.at[pl.ds(off, SCBS)], sidx)
        pltpu.sync_copy(dst_hbm.at[pl.ds(off, SCBS)], didx)
        pltpu.async_copy(z_hbm.at[sidx], buf, sem).wait()
        pltpu.async_copy(buf, acc_sh.at[didx], sem, add=True).wait()
        return carry

    lax.fori_loop(0, EPW // SCBS, chunk, 0)
    plsc.subcore_barrier()

    @pl.when(sid < 10)
    def _():
        pltpu.sync_copy(acc_sh.at[pl.ds(sid * 1000, 1000)],
                        out_hbm.at[c, pl.ds(sid * 1000, 1000)])


def _segsum_es(z, src, dst, w):
    parts = pl.kernel(
        _segsum_es_body,
        out_type=jax.ShapeDtypeStruct((NC, N, w), jnp.float32),
        mesh=plsc.VectorSubcoreMesh(**_SC_MESH),
        scratch_types=[
            pltpu.VMEM_SHARED((N, w), jnp.float32),
            pltpu.VMEM((SCBS,), jnp.int32),
            pltpu.VMEM((SCBS,), jnp.int32),
            pltpu.VMEM((SCBS, w), jnp.float32),
            pltpu.SemaphoreType.DMA,
        ],
    )(z, src, dst)
    return parts[0], parts[1]


NPT = 320
NPAD = NW * NPT
EPWP = EPW + 32 * 16 + 16
_SENT = jnp.int32((NPT << 19))
_M19 = jnp.int32((1 << 19) - 1)


def _route_body(dst_hbm, ebuf_hbm, meta_hbm, dchunk, ebuf, stage, metavm,
                cntv, startsv, ptrv):
    w = _wid()
    lidx = lax.iota(jnp.int32, L)
    z16 = jnp.zeros((L,), jnp.int32)
    pltpu.sync_copy(dst_hbm.at[pl.ds(w * EPW, EPW)], dchunk)

    def fill(i, carry):
        ebuf[pl.ds(i * L, L)] = z16
        return carry

    lax.fori_loop(0, EPWP // L, fill, 0)

    def zero(o, carry):
        cntv[o] = 0
        stage[pl.ds(o * L, L)] = z16
        return carry

    lax.fori_loop(0, 32, zero, 0)

    def count(g, carry):
        dv = dchunk[pl.ds(g * L, L)]
        for j in range(L):
            o = (dv[j] * 6554) >> 21
            cntv[o] = cntv[o] + 1
        return carry

    lax.fori_loop(0, EPW // L, count, 0)

    def prefix(o, pos):
        c = cntv[o]
        startsv[o] = pos
        ptrv[o] = 0
        return (pos + c + 15) & (-16)

    lax.fori_loop(0, 32, prefix, 0)

    def place(g, carry):
        dv = dchunk[pl.ds(g * L, L)]
        for j in range(L):
            d = dv[j]
            o = (d * 6554) >> 21
            dloc = d - o * NPT
            word = (dloc << 19) | (w * EPW + g * L + j)
            cnt = ptrv[o]
            sv = stage[pl.ds(o * L, L)]
            nv = jnp.where(lidx == (cnt & 15), word, sv)
            stage[pl.ds(o * L, L)] = nv

            @pl.when((cnt & 15) == 15)
            def _():
                ebuf[pl.ds(startsv[o] + cnt - 15, L)] = nv

            ptrv[o] = cnt + 1
        return carry

    lax.fori_loop(0, EPW // L, place, 0)

    def flush(o, carry):
        cnt = ptrv[o]

        @pl.when((cnt & 15) > 0)
        def _():
            ebuf[pl.ds(startsv[o] + (cnt & (-16)), L)] = stage[pl.ds(o * L, L)]

        return carry

    lax.fori_loop(0, 32, flush, 0)

    s0 = z16
    s1 = z16
    c0 = z16
    c1 = z16
    for o in range(16):
        s0 = jnp.where(lidx == o, startsv[o], s0)
        s1 = jnp.where(lidx == o, startsv[o + 16], s1)
        c0 = jnp.where(lidx == o, cntv[o], c0)
        c1 = jnp.where(lidx == o, cntv[o + 16], c1)
    metavm[pl.ds(0, L)] = s0
    metavm[pl.ds(L, L)] = s1
    metavm[pl.ds(2 * L, L)] = c0
    metavm[pl.ds(3 * L, L)] = c1

    pltpu.sync_copy(ebuf, ebuf_hbm.at[pl.ds(w * EPWP, EPWP)])
    pltpu.sync_copy(metavm, meta_hbm.at[pl.ds(w * 64, 64)])


def _segmax_body(h_hbm, selfh_hbm, ebuf_hbm, m5_hbm, out_hbm, cnt_hbm,
                 agg, cntb, tb, wbuf, idxbuf, rowbuf, sem):
    w = _wid()
    lo = w * NPT
    lidx = lax.iota(jnp.int32, L)
    one0 = jnp.where(lidx == 0, 1.0, 0.0)
    zf16 = jnp.zeros((L,), jnp.float32)
    pltpu.sync_copy(selfh_hbm.at[pl.ds(lo, NPT)], agg)

    def zcnt(i, carry):
        cntb[i, pl.ds(0, L)] = zf16
        return carry

    lax.fori_loop(0, NPT, zcnt, 0)

    def srctile(t, carry):
        pltpu.sync_copy(m5_hbm.at[pl.ds((t * NW + w) * L, L)], tb)
        tv = tb[pl.ds(0, L)]
        st = pl.multiple_of(tv[0], L)
        ln = tv[1]

        def batch(b, c2):
            off = st + b * L
            pltpu.sync_copy(ebuf_hbm.at[pl.ds(t * EPWP + off, L)], wbuf)
            wv = wbuf[pl.ds(0, L)]
            idxbuf[pl.ds(0, L)] = wv & _M19
            pltpu.async_copy(h_hbm.at[idxbuf], rowbuf, sem).wait()
            for j in range(L):
                dloc = wv[j] >> 19

                @pl.when(b * L + j < ln)
                def _():
                    for c in range(H // L):
                        s = pl.ds(c * L, L)
                        agg[dloc, s] = jnp.maximum(agg[dloc, s], rowbuf[j, s])
                    cntb[dloc, pl.ds(0, L)] = cntb[dloc, pl.ds(0, L)] + one0
            return c2

        lax.fori_loop(0, (ln + L - 1) // L, batch, 0)
        return carry

    lax.fori_loop(0, NW, srctile, 0)
    pltpu.sync_copy(agg, out_hbm.at[pl.ds(lo, NPT)])
    pltpu.sync_copy(cntb, cnt_hbm.at[pl.ds(lo, NPT)])


def _segmax(h, selfh_pad, dst):
    ebufs, meta = pl.kernel(
        _route_body,
        out_type=(jax.ShapeDtypeStruct((NW * EPWP,), jnp.int32),
                  jax.ShapeDtypeStruct((NW * 64,), jnp.int32)),
        mesh=plsc.VectorSubcoreMesh(**_SC_MESH),
        scratch_types=[
            pltpu.VMEM((EPW,), jnp.int32),
            pltpu.VMEM((EPWP,), jnp.int32),
            pltpu.VMEM((32 * L,), jnp.int32),
            pltpu.VMEM((64,), jnp.int32),
            pltpu.SMEM((32,), jnp.int32),
            pltpu.SMEM((32,), jnp.int32),
            pltpu.SMEM((32,), jnp.int32),
        ],
    )(dst)
    m = meta.reshape(NW, 2, 32)
    m5 = jnp.zeros((NW * NW, L), jnp.int32)
    m5 = m5.at[:, 0].set(m[:, 0, :].reshape(-1))
    m5 = m5.at[:, 1].set(m[:, 1, :].reshape(-1))
    return pl.kernel(
        _segmax_body,
        out_type=(jax.ShapeDtypeStruct((NPAD, H), jnp.float32),
                  jax.ShapeDtypeStruct((NPAD, L), jnp.float32)),
        mesh=plsc.VectorSubcoreMesh(**_SC_MESH),
        scratch_types=[
            pltpu.VMEM((NPT, H), jnp.float32),
            pltpu.VMEM((NPT, L), jnp.float32),
            pltpu.VMEM((L,), jnp.int32),
            pltpu.VMEM((L,), jnp.int32),
            pltpu.VMEM((L,), jnp.int32),
            pltpu.VMEM((L, H), jnp.float32),
            pltpu.SemaphoreType.DMA,
        ],
    )(h, selfh_pad, ebufs, m5.reshape(-1))


DCBS = 2000


def _deg_body(dst_hbm, out_hbm, cnt, dchunk):
    w = _wid()

    def z16(i, carry):
        cnt[pl.ds(i * L, L)] = jnp.zeros((L,), jnp.float32)
        return carry

    lax.fori_loop(0, N // L + 1, z16, 0)
    ones = jnp.ones((L,), jnp.float32)

    def chunk(i, carry):
        off = w * EPW + i * DCBS
        pltpu.sync_copy(dst_hbm.at[pl.ds(off, DCBS)], dchunk)

        def grp(j, cc):
            idx = dchunk[pl.ds(j * L, L)]
            plsc.addupdate_scatter(cnt, [idx], ones)
            return cc

        lax.fori_loop(0, DCBS // L, grp, 0)
        return carry

    lax.fori_loop(0, EPW // DCBS, chunk, 0)
    pltpu.sync_copy(cnt, out_hbm.at[w])


def _deg_counts(dst):
    return pl.kernel(
        _deg_body,
        out_type=jax.ShapeDtypeStruct((NW, N + L), jnp.float32),
        mesh=plsc.VectorSubcoreMesh(**_SC_MESH),
        scratch_types=[
            pltpu.VMEM((N + L,), jnp.float32),
            pltpu.VMEM((DCBS,), jnp.int32),
        ],
    )(dst)


BN = 1000


def _nspec(w):
    return pl.BlockSpec((BN, w), lambda i: (i, 0))


def _wspec(r, c):
    return pl.BlockSpec((r, c), lambda i: (0, 0))


def _pre_body(x_ref, p_ref, w1a, w1b, b1, w2, b2, u_ref, pw_ref, sh_ref):
    xw = jnp.dot(x_ref[...], w1a[...], preferred_element_type=jnp.float32) + b1[...]
    pw = jnp.dot(p_ref[...], w1b[...], preferred_element_type=jnp.float32)
    u_ref[...] = xw + pw
    pw_ref[...] = pw
    sh_ref[...] = jnp.dot(jnp.maximum(xw, 0.0), w2[...],
                          preferred_element_type=jnp.float32) + b2[...]


def _pre(x, pos8, w1a, w1b8, b1, w2, b2):
    return pl.pallas_call(
        _pre_body,
        grid=(N // BN,),
        in_specs=[_nspec(D), _nspec(8), _wspec(D, H), _wspec(8, H),
                  _wspec(1, H), _wspec(H, H), _wspec(1, H)],
        out_specs=(_nspec(H), _nspec(H), _nspec(H)),
        out_shape=(jax.ShapeDtypeStruct((N, H), jnp.float32),
                   jax.ShapeDtypeStruct((N, H), jnp.float32),
                   jax.ShapeDtypeStruct((N, H), jnp.float32)),
    )(x, pos8, w1a, w1b8, b1.reshape(1, H), w2, b2.reshape(1, H))


def _dis_of(cnt_ref):
    return lax.rsqrt(cnt_ref[...][:, 0:1] + 1.0)


def _gmlp_body(agg_ref, cnt_ref, gw1, gb1, gw2, gb2, gw3, gb3, cw1,
               z1l_ref, z1r_ref):
    dis = _dis_of(cnt_ref)
    g = jnp.maximum(jnp.dot(agg_ref[...], gw1[...], preferred_element_type=jnp.float32) + gb1[...], 0.0)
    g = jnp.maximum(jnp.dot(g, gw2[...], preferred_element_type=jnp.float32) + gb2[...], 0.0)
    h0 = jnp.dot(g, gw3[...], preferred_element_type=jnp.float32) + gb3[...]
    z1 = dis * jnp.dot(h0, cw1[...], preferred_element_type=jnp.float32)
    z1l_ref[...] = z1[:, : H // 2]
    z1r_ref[...] = z1[:, H // 2 :]


def _gmlp(agg, cnts, gW1, gb1, gW2, gb2, gW3, gb3, cW1):
    return pl.pallas_call(
        _gmlp_body,
        grid=(N // BN,),
        in_specs=[_nspec(H), _nspec(L), _wspec(H, H // 2), _wspec(1, H // 2),
                  _wspec(H // 2, 2 * H), _wspec(1, 2 * H),
                  _wspec(2 * H, H), _wspec(1, H), _wspec(H, H)],
        out_specs=(_nspec(H // 2), _nspec(H // 2)),
        out_shape=(jax.ShapeDtypeStruct((N, H // 2), jnp.float32),
                   jax.ShapeDtypeStruct((N, H // 2), jnp.float32)),
    )(agg, cnts, gW1, gb1.reshape(1, -1), gW2, gb2.reshape(1, -1),
      gW3, gb3.reshape(1, -1), cW1)


def _comb1_body(a1l_ref, a1r_ref, cnt_ref, b1, cw2, z2_ref):
    dis = _dis_of(cnt_ref)
    acc = jnp.concatenate([a1l_ref[...], a1r_ref[...]], axis=1)
    h1 = jnp.maximum(dis * acc + b1[...], 0.0)
    z2_ref[...] = dis * jnp.dot(h1, cw2[...], preferred_element_type=jnp.float32)


def _comb1(a1l, a1r, cnts, cb1, cW2):
    return pl.pallas_call(
        _comb1_body,
        grid=(N // BN,),
        in_specs=[_nspec(H // 2), _nspec(H // 2), _nspec(L),
                  _wspec(1, H), _wspec(H, OUT)],
        out_specs=_nspec(OUT),
        out_shape=jax.ShapeDtypeStruct((N, OUT), jnp.float32),
    )(a1l, a1r, cnts, cb1.reshape(1, H), cW2)


def _comb2_body(p0_ref, p1_ref, z2_ref, cnt_ref, b2, h2_ref):
    dis = _dis_of(cnt_ref)
    acc = p0_ref[...] + p1_ref[...] - z2_ref[...]
    h2_ref[...] = jnp.maximum(dis * acc + b2[...], 0.0)


def _comb2(p0, p1, z2, cnts, cb2):
    return pl.pallas_call(
        _comb2_body,
        grid=(N // BN,),
        in_specs=[_nspec(OUT), _nspec(OUT), _nspec(OUT), _nspec(L),
                  _wspec(1, OUT)],
        out_specs=_nspec(OUT),
        out_shape=jax.ShapeDtypeStruct((N, OUT), jnp.float32),
    )(p0, p1, z2, cnts, cb2.reshape(1, OUT))


def _edge_mlp_body(a_ref, w_ref, b_ref, o_ref):
    a = jnp.maximum(a_ref[...], 0.0)
    o_ref[...] = jnp.dot(a, w_ref[...], preferred_element_type=jnp.float32) + b_ref[...]


def _edge_mlp(a, w, b):
    e = a.shape[0]
    grid = e // EBLK
    return pl.pallas_call(
        _edge_mlp_body,
        grid=(grid,),
        in_specs=[
            pl.BlockSpec((EBLK, H), lambda i: (i, 0)),
            pl.BlockSpec((H, H), lambda i: (0, 0)),
            pl.BlockSpec((1, H), lambda i: (0, 0)),
        ],
        out_specs=pl.BlockSpec((EBLK, H), lambda i: (i, 0)),
        out_shape=jax.ShapeDtypeStruct((e, H), jnp.float32),
    )(a, w, b.reshape(1, H))


def kernel(x, pos, edge_index, lW1, lb1, lW2, lb2, gW1, gb1, gW2, gb2, gW3, gb3, cW1, cb1, cW2, cb2):
    src = edge_index[0]
    dst = edge_index[1]

    pos8 = jnp.pad(pos, ((0, 0), (0, 5)))
    w1b8 = jnp.pad(lW1[D:], ((0, 5), (0, 0)))
    u, pw, self_h = _pre(x, pos8, lW1[:D], w1b8, lb1, lW2, lb2)

    r_e = _edge_gather(u, pw, src, dst)
    h_e = _edge_mlp(r_e, lW2, lb2)

    self_h_pad = jnp.pad(self_h, ((0, NPAD - N), (0, 0)))
    aggp, cntsp = _segmax(h_e, self_h_pad, dst)
    agg = aggp[:N]
    cnts = cntsp[:N]

    z1l, z1r = _gmlp(agg, cnts, gW1, gb1, gW2, gb2, gW3, gb3, cW1)
    acc1l, acc1r = _segsum(z1l, z1r, src, dst, H // 2)
    z2 = _comb1(acc1l, acc1r, cnts, cb1, cW2)
    p0, p1 = _segsum_es(z2, src, dst, OUT)
    return _comb2(p0, p1, z2, cnts, cb2)
